# baseline (speedup 1.0000x reference)
import functools

import jax
import jax.numpy as jnp
from jax import lax
from jax.experimental import pallas as pl
from jax.experimental.pallas import tpu as pltpu
from jax.experimental.pallas import tpu_sc as plsc

N_NODES = 10000
N_EDGES = 320000
N_GRAPHS = 200
G_PAD = 256
V_DIM = 128
E_DIM = 16
H_DIM = 128
K_HEAD = 4

NC = 2
NS = 16
NW = NC * NS
EDGES_PER_TILE = N_EDGES // NW
CHUNK = 40
NCHUNK = EDGES_PER_TILE // CHUNK
ACC_ROWS = N_NODES
ZROWS = 640

BLK = 1000
NBLK = N_NODES // BLK


def _ek_body(e_ref, w_ref, b_ref, out_ref):
    out_ref[...] = jnp.dot(e_ref[...], w_ref[...],
                           preferred_element_type=jnp.float32) + b_ref[...]


def _run_ek(e, wT, b):
    blk = 6400
    return pl.pallas_call(
        _ek_body,
        grid=(N_EDGES // blk,),
        in_specs=[
            pl.BlockSpec((blk, E_DIM), lambda i: (i, 0)),
            pl.BlockSpec((E_DIM, 128), lambda i: (0, 0)),
            pl.BlockSpec((1, 128), lambda i: (0, 0)),
        ],
        out_specs=pl.BlockSpec((blk, 128), lambda i: (i, 0)),
        out_shape=jax.ShapeDtypeStruct((N_EDGES, 128), jnp.float32),
    )(e, wT, b)


def _edge_sc(ek_hbm, v_hbm, src_hbm, dst_hbm, out_hbm,
             vr0, vr1, ekr0, ekr1, sb0, sb1, sb2, sb3, db0, db1, db2, db3,
             semg0, semg1, seme0, seme1, semc0, semc1,
             si0, si1, si2, si3, acc):
    cid = lax.axis_index("c")
    sid = lax.axis_index("s")
    wid = sid * NC + cid
    ebase = wid * EDGES_PER_TILE
    vrs, ekrs = [vr0, vr1], [ekr0, ekr1]
    sbs, dbs = [sb0, sb1, sb2, sb3], [db0, db1, db2, db3]
    semg, seme, semc = [semg0, semg1], [seme0, seme1], [semc0, semc1]
    semi = [si0, si1, si2, si3]

    def zrow(r, _):
        for c in range(8):
            vr0[r, pl.ds(c * 16, 16)] = jnp.zeros((16,), jnp.float32)
        return 0
    lax.fori_loop(0, CHUNK, zrow, 0)
    nz = jnp.where(sid == NS - 1, 10, ZROWS // CHUNK)

    def zcp(k, _):
        pltpu.sync_copy(vr0, acc.at[pl.ds(sid * ZROWS + k * CHUNK, CHUNK), :])
        return 0
    lax.fori_loop(0, nz, zcp, 0)
    plsc.subcore_barrier()

    def _issue(j, sb, vr, ekr, sg, se):
        pltpu.async_copy(v_hbm.at[sb], vr, sg)
        pltpu.async_copy(ek_hbm.at[pl.ds(ebase + j * CHUNK, CHUNK), :],
                         ekr, se)

    pltpu.sync_copy(src_hbm.at[wid, 0], sb0)
    pltpu.sync_copy(dst_hbm.at[wid, 0], db0)
    _issue(0, sb0, vr0, ekr0, semg0, seme0)
    pltpu.async_copy(src_hbm.at[wid, 1], sb1, si1)
    pltpu.async_copy(dst_hbm.at[wid, 1], db1, si1)

    def _step(i, p, q):
        q1, q2 = (q + 1) % 4, (q + 2) % 4
        pltpu.make_async_copy(v_hbm.at[pl.ds(0, CHUNK), :],
                              vrs[p], semg[p]).wait()
        pltpu.make_async_copy(ek_hbm.at[pl.ds(0, CHUNK), :],
                              ekrs[p], seme[p]).wait()

        @pl.when(i + 1 < NCHUNK)
        def _nxt():
            pltpu.make_async_copy(src_hbm.at[0, 0], sbs[q1], semi[q1]).wait()
            pltpu.make_async_copy(dst_hbm.at[0, 0], dbs[q1], semi[q1]).wait()
            @pl.when(i >= 1)
            def _dr():
                pltpu.make_async_copy(vrs[1 - p],
                                      acc.at[pl.ds(0, CHUNK), :],
                                      semc[1 - p]).wait()
            _issue(i + 1, sbs[q1], vrs[1 - p], ekrs[1 - p],
                   semg[1 - p], seme[1 - p])

        def row(r4, _):
            for u in range(4):
                r = r4 * 4 + u
                for c in range(8):
                    sl = pl.ds(c * 16, 16)
                    prod = vrs[p][r, sl] * ekrs[p][r, sl]
                    vrs[p][r, sl] = jnp.maximum(prod, prod * 0.1)
            return 0
        lax.fori_loop(0, CHUNK // 4, row, 0)

        pltpu.async_copy(vrs[p], acc.at[dbs[q]], semc[p], add=True)

        @pl.when(i + 2 < NCHUNK)
        def _pf():
            pltpu.async_copy(src_hbm.at[wid, i + 2], sbs[q2], semi[q2])
            pltpu.async_copy(dst_hbm.at[wid, i + 2], dbs[q2], semi[q2])

    def quad(i4, _):
        i = i4 * 4
        _step(i, 0, 0)
        _step(i + 1, 1, 1)
        _step(i + 2, 0, 2)
        _step(i + 3, 1, 3)
        return 0
    lax.fori_loop(0, NCHUNK // 4, quad, 0)
    _step(NCHUNK - 2, 0, 0)
    _step(NCHUNK - 1, 1, 1)

    pltpu.make_async_copy(vrs[0], acc.at[pl.ds(0, CHUNK), :], semc[0]).wait()
    pltpu.make_async_copy(vrs[1], acc.at[pl.ds(0, CHUNK), :], semc[1]).wait()

    plsc.subcore_barrier()

    nzo = jnp.where(sid == NS - 1, 10, ZROWS // CHUNK)

    def ocp(k, _):
        r0 = sid * ZROWS + k * CHUNK
        pltpu.sync_copy(acc.at[pl.ds(r0, CHUNK), :],
                        out_hbm.at[cid, pl.ds(r0, CHUNK), :])
        return 0
    lax.fori_loop(0, nzo, ocp, 0)


def _run_edge(ek, v, src3d, dst3d):
    mesh = plsc.VectorSubcoreMesh(core_axis_name="c", subcore_axis_name="s")
    fn = functools.partial(
        pl.kernel,
        mesh=mesh,
        out_type=jax.ShapeDtypeStruct((NC, N_NODES, V_DIM), jnp.float32),
        scratch_types=(
            [pltpu.VMEM((CHUNK, V_DIM), jnp.float32)] * 4
            + [pltpu.VMEM((CHUNK,), jnp.int32)] * 8
            + [pltpu.SemaphoreType.DMA] * 10
            + [pltpu.VMEM_SHARED((ACC_ROWS, V_DIM), jnp.float32)]
        ),
    )(_edge_sc)
    return fn(ek, v, src3d, dst3d)


def _node_body(v_ref, svep_ref, seg_ref, s_ref,
               waT_ref, ba_ref, wdT_ref, bd_ref, wbT_ref, bb_ref,
               cw_ref, cb_ref,
               aT_ref, ab_ref, cT_ref, cbias_ref, bT_ref, bbias_ref,
               e1T_ref, e2T_ref, eb_ref,
               gmAT_ref, gmAb_ref, gmBT_ref, gmBb_ref,
               gmIT_ref, gmIb_ref, gmHT_ref, gmHb_ref,
               gsAT_ref, gsAb_ref, gsBT_ref, gsBb_ref,
               gsIT_ref, gsIb_ref, gsHT_ref, gsHb_ref,
               outv_ref, outs_ref,
               num_acc, den_acc):
    i = pl.program_id(0)

    @pl.when(i == 0)
    def _init():
        num_acc[...] = jnp.zeros((K_HEAD * G_PAD, H_DIM), jnp.float32)
        den_acc[...] = jnp.zeros((8, G_PAD), jnp.float32)

    seg = seg_ref[0, 0, :]
    gids = lax.broadcasted_iota(jnp.int32, (BLK, G_PAD), 1)
    onehot = (seg[:, None] == gids).astype(jnp.float32)

    v = v_ref[...]
    s = s_ref[...]

    P = jnp.tanh(jnp.dot(v, waT_ref[...],
                         preferred_element_type=jnp.float32) + ba_ref[...])
    Dv = jnp.dot(v, wdT_ref[...],
                 preferred_element_type=jnp.float32) + bd_ref[...]
    Q = jnp.tanh(jnp.dot(s, wbT_ref[...],
                         preferred_element_type=jnp.float32) + bb_ref[...])
    Q = Q * cw_ref[...]
    Qseg = jnp.dot(onehot, Q, preferred_element_type=jnp.float32)
    prod = P * Qseg
    cb = cb_ref[...]
    for h in range(K_HEAD):
        sl = slice(h * H_DIM, (h + 1) * H_DIM)
        a = jnp.sum(prod[:, sl], axis=1) + cb[0, h]
        ea = jnp.exp(a)
        Wh = onehot * ea[:, None]
        numc = lax.dot_general(Wh, Dv[:, sl],
                               (((0,), (0,)), ((), ())),
                               preferred_element_type=jnp.float32)
        rs = pl.ds(h * G_PAD, G_PAD)
        num_acc[rs, :] = num_acc[rs, :] + numc
        den_acc[h, :] = den_acc[h, :] + jnp.sum(Wh, axis=0)

    sve = svep_ref[0] + svep_ref[1]
    tsc = jnp.tanh(jnp.dot(s, cT_ref[...],
                           preferred_element_type=jnp.float32) + cbias_ref[...])
    s2m = jnp.dot(onehot, tsc, preferred_element_type=jnp.float32)
    pre = (jnp.dot(sve, e1T_ref[...], preferred_element_type=jnp.float32)
           + jnp.dot(v, e2T_ref[...], preferred_element_type=jnp.float32)
           + eb_ref[...])
    m2m = jnp.maximum(pre, pre * 0.1)
    z = jax.nn.sigmoid(
        jnp.dot(m2m, gmAT_ref[...], preferred_element_type=jnp.float32)
        + gmAb_ref[...]
        + jnp.dot(s2m, gmBT_ref[...], preferred_element_type=jnp.float32)
        + gmBb_ref[...])
    h0 = z * s2m + (1.0 - z) * m2m
    gi = jnp.dot(v, gmIT_ref[...],
                 preferred_element_type=jnp.float32) + gmIb_ref[...]
    gh = jnp.dot(h0, gmHT_ref[...],
                 preferred_element_type=jnp.float32) + gmHb_ref[...]
    r = jax.nn.sigmoid(gi[:, :128] + gh[:, :128])
    zz = jax.nn.sigmoid(gi[:, 128:256] + gh[:, 128:256])
    n = jnp.tanh(gi[:, 256:] + r * gh[:, 256:])
    outv_ref[...] = (1.0 - zz) * n + zz * h0

    @pl.when(i == NBLK - 1)
    def _fin():
        den = den_acc[...]
        hs_list = []
        for h in range(K_HEAD):
            dh = den[h, :]
            dh = jnp.where(dh == 0.0, 1.0, dh)
            hs_list.append(num_acc[pl.ds(h * G_PAD, G_PAD), :] / dh[:, None])
        cat = jnp.concatenate(hs_list, axis=1)
        m2s = jnp.tanh(jnp.dot(cat, bT_ref[...],
                               preferred_element_type=jnp.float32)
                       + bbias_ref[...])
        s2s = jnp.tanh(jnp.dot(s, aT_ref[...],
                               preferred_element_type=jnp.float32)
                       + ab_ref[...])
        zs = jax.nn.sigmoid(
            jnp.dot(s2s, gsAT_ref[...], preferred_element_type=jnp.float32)
            + gsAb_ref[...]
            + jnp.dot(m2s, gsBT_ref[...], preferred_element_type=jnp.float32)
            + gsBb_ref[...])
        hs = zs * m2s + (1.0 - zs) * s2s
        gi2 = jnp.dot(s, gsIT_ref[...],
                      preferred_element_type=jnp.float32) + gsIb_ref[...]
        gh2 = jnp.dot(hs, gsHT_ref[...],
                      preferred_element_type=jnp.float32) + gsHb_ref[...]
        r2 = jax.nn.sigmoid(gi2[:, :128] + gh2[:, :128])
        zz2 = jax.nn.sigmoid(gi2[:, 128:256] + gh2[:, 128:256])
        n2 = jnp.tanh(gi2[:, 256:] + r2 * gh2[:, 256:])
        res = (1.0 - zz2) * n2 + zz2 * hs
        outs_ref[...] = res[:N_GRAPHS, :]


def _run_node(v, svep, seg3, s_pad, weights):
    full = lambda shape: pl.BlockSpec(shape, lambda i: tuple(0 for _ in shape))
    w_specs = [full(w.shape) for w in weights]
    return pl.pallas_call(
        _node_body,
        grid=(NBLK,),
        in_specs=[
            pl.BlockSpec((BLK, V_DIM), lambda i: (i, 0)),
            pl.BlockSpec((NC, BLK, V_DIM), lambda i: (0, i, 0)),
            pl.BlockSpec((1, 1, BLK), lambda i: (i, 0, 0)),
            full((G_PAD, V_DIM)),
        ] + w_specs,
        out_specs=[
            pl.BlockSpec((BLK, H_DIM), lambda i: (i, 0)),
            pl.BlockSpec((N_GRAPHS, H_DIM), lambda i: (0, 0)),
        ],
        out_shape=[
            jax.ShapeDtypeStruct((N_NODES, H_DIM), jnp.float32),
            jax.ShapeDtypeStruct((N_GRAPHS, H_DIM), jnp.float32),
        ],
        scratch_shapes=[
            pltpu.VMEM((K_HEAD * G_PAD, H_DIM), jnp.float32),
            pltpu.VMEM((8, G_PAD), jnp.float32),
        ],
    )(v, svep, seg3, s_pad, *weights)


def kernel(edge_index, node2graph, v, e, s, params):
    f32 = jnp.float32

    kw, kb = params['K']['w'], params['K']['b']
    ek = _run_ek(e, kw.T, kb[None, :])

    src3d = edge_index[0].reshape(NW, NCHUNK, CHUNK)
    dst3d = edge_index[1].reshape(NW, NCHUNK, CHUNK)
    svep = _run_edge(ek, v, src3d, dst3d)

    seg3 = node2graph.reshape(NBLK, 1, BLK)
    s_pad = jnp.zeros((G_PAD, V_DIM), f32).at[:N_GRAPHS].set(s)

    heads = params['heads']
    waT = jnp.concatenate([hp['A']['w'] for hp in heads], axis=0).T
    ba = jnp.concatenate([hp['A']['b'] for hp in heads])[None, :]
    wdT = jnp.concatenate([hp['D']['w'] for hp in heads], axis=0).T
    bd = jnp.concatenate([hp['D']['b'] for hp in heads])[None, :]
    wbT = jnp.concatenate([hp['B']['w'] for hp in heads], axis=0).T
    bb = jnp.concatenate([hp['B']['b'] for hp in heads])[None, :]
    cw = jnp.concatenate([hp['C']['w'][0] for hp in heads])[None, :]
    cb = jnp.zeros((1, 128), f32)
    for h in range(K_HEAD):
        cb = cb.at[0, h].set(heads[h]['C']['b'][0])

    gm, gs = params['gm'], params['gs']
    weights = [
        waT, ba, wdT, bd, wbT, bb, cw, cb,
        params['A']['w'].T, params['A']['b'][None, :],
        params['C']['w'].T, params['C']['b'][None, :],
        params['B']['w'].T, params['B']['b'][None, :],
        params['E']['w'][:, :128].T, params['E']['w'][:, 128:].T,
        params['E']['b'][None, :],
        gm['A']['w'].T, gm['A']['b'][None, :],
        gm['B']['w'].T, gm['B']['b'][None, :],
        gm['w_ih'].T, gm['b_ih'][None, :],
        gm['w_hh'].T, gm['b_hh'][None, :],
        gs['A']['w'].T, gs['A']['b'][None, :],
        gs['B']['w'].T, gs['B']['b'][None, :],
        gs['w_ih'].T, gs['b_ih'][None, :],
        gs['w_hh'].T, gs['b_hh'][None, :],
    ]
    update_v, update_s = _run_node(v, svep, seg3, s_pad, weights)
    return update_v, update_s


# speedup vs baseline: 1.1361x; 1.1361x over previous
import functools

import jax
import jax.numpy as jnp
from jax import lax
from jax.experimental import pallas as pl
from jax.experimental.pallas import tpu as pltpu
from jax.experimental.pallas import tpu_sc as plsc

N_NODES = 10000
N_EDGES = 320000
N_GRAPHS = 200
G_PAD = 256
V_DIM = 128
E_DIM = 16
H_DIM = 128
K_HEAD = 4

NC = 2
NS = 16
NW = NC * NS
EDGES_PER_TILE = N_EDGES // NW
CHUNK = 80
NCHUNK = EDGES_PER_TILE // CHUNK
ACC_ROWS = N_NODES
ZROWS = 640

BLK = 1000
NBLK = N_NODES // BLK


def _ek_body(e_ref, w_ref, b_ref, out_ref):
    out_ref[...] = jnp.dot(e_ref[...], w_ref[...],
                           preferred_element_type=jnp.float32) + b_ref[...]


def _run_ek(e, wT, b):
    blk = 6400
    return pl.pallas_call(
        _ek_body,
        grid=(N_EDGES // blk,),
        in_specs=[
            pl.BlockSpec((blk, E_DIM), lambda i: (i, 0)),
            pl.BlockSpec((E_DIM, 128), lambda i: (0, 0)),
            pl.BlockSpec((1, 128), lambda i: (0, 0)),
        ],
        out_specs=pl.BlockSpec((blk, 128), lambda i: (i, 0)),
        out_shape=jax.ShapeDtypeStruct((N_EDGES, 128), jnp.float32),
    )(e, wT, b)


def _edge_sc(ek_hbm, v_hbm, src_hbm, dst_hbm, out_hbm,
             vr0, vr1, ekr0, ekr1, sb0, sb1, sb2, sb3, db0, db1, db2, db3,
             semg0, semg1, seme0, seme1, semc0, semc1,
             si0, si1, si2, si3, acc):
    cid = lax.axis_index("c")
    sid = lax.axis_index("s")
    wid = sid * NC + cid
    ebase = wid * EDGES_PER_TILE
    vrs, ekrs = [vr0, vr1], [ekr0, ekr1]
    sbs, dbs = [sb0, sb1, sb2, sb3], [db0, db1, db2, db3]
    semg, seme, semc = [semg0, semg1], [seme0, seme1], [semc0, semc1]
    semi = [si0, si1, si2, si3]

    def zrow(r, _):
        for c in range(8):
            vr0[r, pl.ds(c * 16, 16)] = jnp.zeros((16,), jnp.float32)
        return 0
    lax.fori_loop(0, CHUNK, zrow, 0)
    nz = jnp.where(sid == NS - 1, (N_NODES - (NS - 1) * ZROWS) // CHUNK,
                   ZROWS // CHUNK)

    def zcp(k, _):
        pltpu.sync_copy(vr0, acc.at[pl.ds(sid * ZROWS + k * CHUNK, CHUNK), :])
        return 0
    lax.fori_loop(0, nz, zcp, 0)
    plsc.subcore_barrier()

    def _issue(j, sb, vr, ekr, sg, se):
        pltpu.async_copy(v_hbm.at[sb], vr, sg)
        pltpu.async_copy(ek_hbm.at[pl.ds(ebase + j * CHUNK, CHUNK), :],
                         ekr, se)

    pltpu.sync_copy(src_hbm.at[wid, 0], sb0)
    pltpu.sync_copy(dst_hbm.at[wid, 0], db0)
    _issue(0, sb0, vr0, ekr0, semg0, seme0)
    pltpu.async_copy(src_hbm.at[wid, 1], sb1, si1)
    pltpu.async_copy(dst_hbm.at[wid, 1], db1, si1)

    def _step(i, p, q):
        q1, q2 = (q + 1) % 4, (q + 2) % 4
        pltpu.make_async_copy(v_hbm.at[pl.ds(0, CHUNK), :],
                              vrs[p], semg[p]).wait()
        pltpu.make_async_copy(ek_hbm.at[pl.ds(0, CHUNK), :],
                              ekrs[p], seme[p]).wait()

        @pl.when(i + 1 < NCHUNK)
        def _nxt():
            pltpu.make_async_copy(src_hbm.at[0, 0], sbs[q1], semi[q1]).wait()
            pltpu.make_async_copy(dst_hbm.at[0, 0], dbs[q1], semi[q1]).wait()
            @pl.when(i >= 1)
            def _dr():
                pltpu.make_async_copy(vrs[1 - p],
                                      acc.at[pl.ds(0, CHUNK), :],
                                      semc[1 - p]).wait()
            _issue(i + 1, sbs[q1], vrs[1 - p], ekrs[1 - p],
                   semg[1 - p], seme[1 - p])

        def row(r4, _):
            for u in range(4):
                r = r4 * 4 + u
                for c in range(8):
                    sl = pl.ds(c * 16, 16)
                    prod = vrs[p][r, sl] * ekrs[p][r, sl]
                    vrs[p][r, sl] = jnp.maximum(prod, prod * 0.1)
            return 0
        lax.fori_loop(0, CHUNK // 4, row, 0)

        pltpu.async_copy(vrs[p], acc.at[dbs[q]], semc[p], add=True)

        @pl.when(i + 2 < NCHUNK)
        def _pf():
            pltpu.async_copy(src_hbm.at[wid, i + 2], sbs[q2], semi[q2])
            pltpu.async_copy(dst_hbm.at[wid, i + 2], dbs[q2], semi[q2])

    def quad(i4, _):
        i = i4 * 4
        _step(i, 0, 0)
        _step(i + 1, 1, 1)
        _step(i + 2, 0, 2)
        _step(i + 3, 1, 3)
        return 0
    lax.fori_loop(0, NCHUNK // 4, quad, 0)
    for t in range((NCHUNK // 4) * 4, NCHUNK):
        _step(t, t % 2, t % 4)

    pltpu.make_async_copy(vrs[0], acc.at[pl.ds(0, CHUNK), :], semc[0]).wait()
    pltpu.make_async_copy(vrs[1], acc.at[pl.ds(0, CHUNK), :], semc[1]).wait()

    plsc.subcore_barrier()

    nzo = jnp.where(sid == NS - 1, (N_NODES - (NS - 1) * ZROWS) // CHUNK,
                    ZROWS // CHUNK)

    def ocp(k, _):
        r0 = sid * ZROWS + k * CHUNK
        pltpu.sync_copy(acc.at[pl.ds(r0, CHUNK), :],
                        out_hbm.at[cid, pl.ds(r0, CHUNK), :])
        return 0
    lax.fori_loop(0, nzo, ocp, 0)


def _run_edge(ek, v, src3d, dst3d):
    mesh = plsc.VectorSubcoreMesh(core_axis_name="c", subcore_axis_name="s")
    fn = functools.partial(
        pl.kernel,
        mesh=mesh,
        out_type=jax.ShapeDtypeStruct((NC, N_NODES, V_DIM), jnp.float32),
        scratch_types=(
            [pltpu.VMEM((CHUNK, V_DIM), jnp.float32)] * 4
            + [pltpu.VMEM((CHUNK,), jnp.int32)] * 8
            + [pltpu.SemaphoreType.DMA] * 10
            + [pltpu.VMEM_SHARED((ACC_ROWS, V_DIM), jnp.float32)]
        ),
    )(_edge_sc)
    return fn(ek, v, src3d, dst3d)


def _node_body(v_ref, svep_ref, seg_ref, s_ref,
               waT_ref, ba_ref, wdT_ref, bd_ref, wbT_ref, bb_ref,
               cw_ref, cb_ref,
               aT_ref, ab_ref, cT_ref, cbias_ref, bT_ref, bbias_ref,
               e1T_ref, e2T_ref, eb_ref,
               gmAT_ref, gmAb_ref, gmBT_ref, gmBb_ref,
               gmIT_ref, gmIb_ref, gmHT_ref, gmHb_ref,
               gsAT_ref, gsAb_ref, gsBT_ref, gsBb_ref,
               gsIT_ref, gsIb_ref, gsHT_ref, gsHb_ref,
               outv_ref, outs_ref,
               num_acc, den_acc):
    i = pl.program_id(0)

    @pl.when(i == 0)
    def _init():
        num_acc[...] = jnp.zeros((K_HEAD * G_PAD, H_DIM), jnp.float32)
        den_acc[...] = jnp.zeros((8, G_PAD), jnp.float32)

    seg = seg_ref[0, 0, :]
    gids = lax.broadcasted_iota(jnp.int32, (BLK, G_PAD), 1)
    onehot = (seg[:, None] == gids).astype(jnp.float32)

    v = v_ref[...]
    s = s_ref[...]

    P = jnp.tanh(jnp.dot(v, waT_ref[...],
                         preferred_element_type=jnp.float32) + ba_ref[...])
    Dv = jnp.dot(v, wdT_ref[...],
                 preferred_element_type=jnp.float32) + bd_ref[...]
    Q = jnp.tanh(jnp.dot(s, wbT_ref[...],
                         preferred_element_type=jnp.float32) + bb_ref[...])
    Q = Q * cw_ref[...]
    Qseg = jnp.dot(onehot, Q, preferred_element_type=jnp.float32)
    prod = P * Qseg
    cb = cb_ref[...]
    for h in range(K_HEAD):
        sl = slice(h * H_DIM, (h + 1) * H_DIM)
        a = jnp.sum(prod[:, sl], axis=1) + cb[0, h]
        ea = jnp.exp(a)
        Wh = onehot * ea[:, None]
        numc = lax.dot_general(Wh, Dv[:, sl],
                               (((0,), (0,)), ((), ())),
                               preferred_element_type=jnp.float32)
        rs = pl.ds(h * G_PAD, G_PAD)
        num_acc[rs, :] = num_acc[rs, :] + numc
        den_acc[h, :] = den_acc[h, :] + jnp.sum(Wh, axis=0)

    sve = svep_ref[0] + svep_ref[1]
    tsc = jnp.tanh(jnp.dot(s, cT_ref[...],
                           preferred_element_type=jnp.float32) + cbias_ref[...])
    s2m = jnp.dot(onehot, tsc, preferred_element_type=jnp.float32)
    pre = (jnp.dot(sve, e1T_ref[...], preferred_element_type=jnp.float32)
           + jnp.dot(v, e2T_ref[...], preferred_element_type=jnp.float32)
           + eb_ref[...])
    m2m = jnp.maximum(pre, pre * 0.1)
    z = jax.nn.sigmoid(
        jnp.dot(m2m, gmAT_ref[...], preferred_element_type=jnp.float32)
        + gmAb_ref[...]
        + jnp.dot(s2m, gmBT_ref[...], preferred_element_type=jnp.float32)
        + gmBb_ref[...])
    h0 = z * s2m + (1.0 - z) * m2m
    gi = jnp.dot(v, gmIT_ref[...],
                 preferred_element_type=jnp.float32) + gmIb_ref[...]
    gh = jnp.dot(h0, gmHT_ref[...],
                 preferred_element_type=jnp.float32) + gmHb_ref[...]
    r = jax.nn.sigmoid(gi[:, :128] + gh[:, :128])
    zz = jax.nn.sigmoid(gi[:, 128:256] + gh[:, 128:256])
    n = jnp.tanh(gi[:, 256:] + r * gh[:, 256:])
    outv_ref[...] = (1.0 - zz) * n + zz * h0

    @pl.when(i == NBLK - 1)
    def _fin():
        den = den_acc[...]
        hs_list = []
        for h in range(K_HEAD):
            dh = den[h, :]
            dh = jnp.where(dh == 0.0, 1.0, dh)
            hs_list.append(num_acc[pl.ds(h * G_PAD, G_PAD), :] / dh[:, None])
        cat = jnp.concatenate(hs_list, axis=1)
        m2s = jnp.tanh(jnp.dot(cat, bT_ref[...],
                               preferred_element_type=jnp.float32)
                       + bbias_ref[...])
        s2s = jnp.tanh(jnp.dot(s, aT_ref[...],
                               preferred_element_type=jnp.float32)
                       + ab_ref[...])
        zs = jax.nn.sigmoid(
            jnp.dot(s2s, gsAT_ref[...], preferred_element_type=jnp.float32)
            + gsAb_ref[...]
            + jnp.dot(m2s, gsBT_ref[...], preferred_element_type=jnp.float32)
            + gsBb_ref[...])
        hs = zs * m2s + (1.0 - zs) * s2s
        gi2 = jnp.dot(s, gsIT_ref[...],
                      preferred_element_type=jnp.float32) + gsIb_ref[...]
        gh2 = jnp.dot(hs, gsHT_ref[...],
                      preferred_element_type=jnp.float32) + gsHb_ref[...]
        r2 = jax.nn.sigmoid(gi2[:, :128] + gh2[:, :128])
        zz2 = jax.nn.sigmoid(gi2[:, 128:256] + gh2[:, 128:256])
        n2 = jnp.tanh(gi2[:, 256:] + r2 * gh2[:, 256:])
        res = (1.0 - zz2) * n2 + zz2 * hs
        outs_ref[...] = res[:N_GRAPHS, :]


def _run_node(v, svep, seg3, s_pad, weights):
    full = lambda shape: pl.BlockSpec(shape, lambda i: tuple(0 for _ in shape))
    w_specs = [full(w.shape) for w in weights]
    return pl.pallas_call(
        _node_body,
        grid=(NBLK,),
        in_specs=[
            pl.BlockSpec((BLK, V_DIM), lambda i: (i, 0)),
            pl.BlockSpec((NC, BLK, V_DIM), lambda i: (0, i, 0)),
            pl.BlockSpec((1, 1, BLK), lambda i: (i, 0, 0)),
            full((G_PAD, V_DIM)),
        ] + w_specs,
        out_specs=[
            pl.BlockSpec((BLK, H_DIM), lambda i: (i, 0)),
            pl.BlockSpec((N_GRAPHS, H_DIM), lambda i: (0, 0)),
        ],
        out_shape=[
            jax.ShapeDtypeStruct((N_NODES, H_DIM), jnp.float32),
            jax.ShapeDtypeStruct((N_GRAPHS, H_DIM), jnp.float32),
        ],
        scratch_shapes=[
            pltpu.VMEM((K_HEAD * G_PAD, H_DIM), jnp.float32),
            pltpu.VMEM((8, G_PAD), jnp.float32),
        ],
    )(v, svep, seg3, s_pad, *weights)


def kernel(edge_index, node2graph, v, e, s, params):
    f32 = jnp.float32

    kw, kb = params['K']['w'], params['K']['b']
    ek = _run_ek(e, kw.T, kb[None, :])

    src3d = edge_index[0].reshape(NW, NCHUNK, CHUNK)
    dst3d = edge_index[1].reshape(NW, NCHUNK, CHUNK)
    svep = _run_edge(ek, v, src3d, dst3d)

    seg3 = node2graph.reshape(NBLK, 1, BLK)
    s_pad = jnp.zeros((G_PAD, V_DIM), f32).at[:N_GRAPHS].set(s)

    heads = params['heads']
    waT = jnp.concatenate([hp['A']['w'] for hp in heads], axis=0).T
    ba = jnp.concatenate([hp['A']['b'] for hp in heads])[None, :]
    wdT = jnp.concatenate([hp['D']['w'] for hp in heads], axis=0).T
    bd = jnp.concatenate([hp['D']['b'] for hp in heads])[None, :]
    wbT = jnp.concatenate([hp['B']['w'] for hp in heads], axis=0).T
    bb = jnp.concatenate([hp['B']['b'] for hp in heads])[None, :]
    cw = jnp.concatenate([hp['C']['w'][0] for hp in heads])[None, :]
    cb = jnp.zeros((1, 128), f32)
    for h in range(K_HEAD):
        cb = cb.at[0, h].set(heads[h]['C']['b'][0])

    gm, gs = params['gm'], params['gs']
    weights = [
        waT, ba, wdT, bd, wbT, bb, cw, cb,
        params['A']['w'].T, params['A']['b'][None, :],
        params['C']['w'].T, params['C']['b'][None, :],
        params['B']['w'].T, params['B']['b'][None, :],
        params['E']['w'][:, :128].T, params['E']['w'][:, 128:].T,
        params['E']['b'][None, :],
        gm['A']['w'].T, gm['A']['b'][None, :],
        gm['B']['w'].T, gm['B']['b'][None, :],
        gm['w_ih'].T, gm['b_ih'][None, :],
        gm['w_hh'].T, gm['b_hh'][None, :],
        gs['A']['w'].T, gs['A']['b'][None, :],
        gs['B']['w'].T, gs['B']['b'][None, :],
        gs['w_ih'].T, gs['b_ih'][None, :],
        gs['w_hh'].T, gs['b_hh'][None, :],
    ]
    update_v, update_s = _run_node(v, svep, seg3, s_pad, weights)
    return update_v, update_s
